# baseline (speedup 1.0000x reference)
import jax
import jax.numpy as jnp
from jax.experimental import pallas as pl
from jax.experimental.pallas import tpu as pltpu

B = 2
N = 4096
D = 256
BI = 2048
RB = 512
NI = N // BI


def _top3_tournament(s):
    r = s.shape[0] // 2
    t1 = jnp.minimum(s[:r], s[r:])
    t2 = jnp.maximum(s[:r], s[r:])
    r //= 2
    a1, b1 = t1[:r], t1[r:]
    a2, b2 = t2[:r], t2[r:]
    mx1 = jnp.maximum(a1, b1)
    mn2 = jnp.minimum(a2, b2)
    t1 = jnp.minimum(a1, b1)
    t3 = jnp.maximum(mx1, mn2)
    t2 = jnp.minimum(mx1, mn2)
    while r > 1:
        r //= 2
        a1, b1 = t1[:r], t1[r:]
        a2, b2 = t2[:r], t2[r:]
        a3, b3 = t3[:r], t3[r:]
        mx1 = jnp.maximum(a1, b1)
        mn2 = jnp.minimum(a2, b2)
        mx2 = jnp.maximum(a2, b2)
        mn3 = jnp.minimum(a3, b3)
        t1 = jnp.minimum(a1, b1)
        t2 = jnp.minimum(mx1, mn2)
        t3 = jnp.minimum(jnp.maximum(mx1, mn2), jnp.minimum(mx2, mn3))
    return t1, t2, t3


def _twonn_fused_kernel(xi_ref, xj_ref, o1_ref, o2_ref, sqjh_ref, d_ref):
    i = pl.program_id(1)

    @pl.when(i == 0)
    def _norms():
        xj = xj_ref[0]
        sqjh_ref[:, 0] = 0.5 * jnp.sum(xj * xj, axis=1)

    xi = xi_ref[0]
    xj = xj_ref[0]
    dot = jax.lax.dot_general(
        xj, xi, (((1,), (1,)), ((), ())), preferred_element_type=jnp.float32
    )
    s = sqjh_ref[...] - dot
    t1, m2, m3 = _top3_tournament(s)
    d_ref[0:1, pl.ds(i * BI, BI)] = 2.0 * (m2 - t1)
    d_ref[1:2, pl.ds(i * BI, BI)] = 2.0 * (m3 - t1)

    @pl.when(i == NI - 1)
    def _twonn():
        d1f = d_ref[0:1, :]
        d2f = d_ref[1:2, :]
        tf = 0.5 * (jnp.log(d2f) - jnp.log(d1f))
        sxy = jnp.float32(0.0)
        sxx = jnp.sum(tf * tf)
        ones_nk = jnp.ones((N, 8), jnp.float32)
        for r in range(N // RB):
            tb = jnp.transpose(tf[:, r * RB : (r + 1) * RB])
            less = (tf < tb).astype(jnp.float32)
            rank = jax.lax.dot_general(
                less, ones_nk, (((1,), (0,)), ((), ())),
                preferred_element_type=jnp.float32,
            )[:, 0:1]
            y = jnp.log(jnp.float32(N)) - jnp.log(jnp.float32(N) - rank)
            sxy = sxy + jnp.sum(tb * y)
        o1_ref[...] = jnp.full((1, 8, 128), 1.0, jnp.float32) * sxy
        o2_ref[...] = jnp.full((1, 8, 128), 1.0, jnp.float32) * sxx


def kernel(X):
    o1, o2 = pl.pallas_call(
        _twonn_fused_kernel,
        grid=(B, NI),
        in_specs=[
            pl.BlockSpec((1, BI, D), lambda b, i: (b, i, 0)),
            pl.BlockSpec((1, N, D), lambda b, i: (b, 0, 0)),
        ],
        out_specs=[
            pl.BlockSpec((1, 8, 128), lambda b, i: (b, 0, 0)),
            pl.BlockSpec((1, 8, 128), lambda b, i: (b, 0, 0)),
        ],
        out_shape=[
            jax.ShapeDtypeStruct((B, 8, 128), jnp.float32),
            jax.ShapeDtypeStruct((B, 8, 128), jnp.float32),
        ],
        scratch_shapes=[
            pltpu.VMEM((N, 1), jnp.float32),
            pltpu.VMEM((2, N), jnp.float32),
        ],
        compiler_params=pltpu.CompilerParams(
            dimension_semantics=("parallel", "arbitrary"),
        ),
    )(X, X)
    return o1[:, 0, 0] / o2[:, 0, 0]


# speedup vs baseline: 1.0243x; 1.0243x over previous
import jax
import jax.numpy as jnp
from jax.experimental import pallas as pl
from jax.experimental.pallas import tpu as pltpu

B = 2
N = 4096
D = 256
BI = 2048
RB = 512
NI = N // BI


def _top3_tournament(s):
    r = s.shape[0] // 2
    t1 = jnp.minimum(s[:r], s[r:])
    t2 = jnp.maximum(s[:r], s[r:])
    r //= 2
    a1, b1 = t1[:r], t1[r:]
    a2, b2 = t2[:r], t2[r:]
    mx1 = jnp.maximum(a1, b1)
    mn2 = jnp.minimum(a2, b2)
    t1 = jnp.minimum(a1, b1)
    t3 = jnp.maximum(mx1, mn2)
    t2 = jnp.minimum(mx1, mn2)
    while r > 1:
        r //= 2
        a1, b1 = t1[:r], t1[r:]
        a2, b2 = t2[:r], t2[r:]
        a3, b3 = t3[:r], t3[r:]
        mx1 = jnp.maximum(a1, b1)
        mn2 = jnp.minimum(a2, b2)
        mx2 = jnp.maximum(a2, b2)
        mn3 = jnp.minimum(a3, b3)
        t1 = jnp.minimum(a1, b1)
        t2 = jnp.minimum(mx1, mn2)
        t3 = jnp.minimum(jnp.maximum(mx1, mn2), jnp.minimum(mx2, mn3))
    return t1, t2, t3


def _twonn_fused_kernel(xi_ref, xj_ref, o1_ref, o2_ref, sqjh_ref, d_ref):
    i = pl.program_id(1)

    @pl.when(i == 0)
    def _norms():
        xj = xj_ref[0]
        sqjh_ref[:, 0] = 0.5 * jnp.sum(xj * xj, axis=1)

    xi = xi_ref[0]
    xj = xj_ref[0]
    dot = jax.lax.dot_general(
        xj, xi, (((1,), (1,)), ((), ())), preferred_element_type=jnp.float32
    )
    s = sqjh_ref[...] - dot
    t1, m2, m3 = _top3_tournament(s)
    d_ref[0:1, pl.ds(i * BI, BI)] = 2.0 * (m2 - t1)
    d_ref[1:2, pl.ds(i * BI, BI)] = 2.0 * (m3 - t1)

    @pl.when(i == NI - 1)
    def _twonn():
        d1f = d_ref[0:1, :]
        d2f = d_ref[1:2, :]
        tf = 0.5 * (jnp.log(d2f) - jnp.log(d1f))
        sxy = jnp.float32(0.0)
        sxx = jnp.sum(tf * tf)
        for r in range(N // RB):
            tb = jnp.transpose(tf[:, r * RB : (r + 1) * RB])
            rank = jnp.count_nonzero(tf < tb, axis=1, keepdims=True).astype(
                jnp.float32
            )
            y = jnp.log(jnp.float32(N)) - jnp.log(jnp.float32(N) - rank)
            sxy = sxy + jnp.sum(tb * y)
        o1_ref[...] = jnp.full((1, 8, 128), 1.0, jnp.float32) * sxy
        o2_ref[...] = jnp.full((1, 8, 128), 1.0, jnp.float32) * sxx


def kernel(X):
    o1, o2 = pl.pallas_call(
        _twonn_fused_kernel,
        grid=(B, NI),
        in_specs=[
            pl.BlockSpec((1, BI, D), lambda b, i: (b, i, 0)),
            pl.BlockSpec((1, N, D), lambda b, i: (b, 0, 0)),
        ],
        out_specs=[
            pl.BlockSpec((1, 8, 128), lambda b, i: (b, 0, 0)),
            pl.BlockSpec((1, 8, 128), lambda b, i: (b, 0, 0)),
        ],
        out_shape=[
            jax.ShapeDtypeStruct((B, 8, 128), jnp.float32),
            jax.ShapeDtypeStruct((B, 8, 128), jnp.float32),
        ],
        scratch_shapes=[
            pltpu.VMEM((N, 1), jnp.float32),
            pltpu.VMEM((2, N), jnp.float32),
        ],
        compiler_params=pltpu.CompilerParams(
            dimension_semantics=("parallel", "arbitrary"),
        ),
    )(X, X)
    return o1[:, 0, 0] / o2[:, 0, 0]
